# baseline (speedup 1.0000x reference)
import jax
import jax.numpy as jnp
from jax.experimental import pallas as pl
from jax.experimental.pallas import tpu as pltpu

N = 65536
K = 256
M = 64
BLOCK_N = 1024
NBUF = 8
NSTEPS = N // BLOCK_N


def _mm_body(xa_hbm, xb_hbm, wt_ref, b_ref, o_hbm, xbuf, obuf, insems, outsems):
    xrefs = [xa_hbm, xb_hbm]

    def in_copy(i, s):
        return pltpu.make_async_copy(
            xrefs[i % 2].at[pl.ds(i * BLOCK_N, BLOCK_N), :],
            xbuf.at[s],
            insems.at[s],
        )

    def out_copy(i, s):
        return pltpu.make_async_copy(
            obuf.at[s], o_hbm.at[pl.ds(i * BLOCK_N, BLOCK_N), :], outsems.at[s]
        )

    for i in range(NBUF):
        in_copy(i, i).start()
    for i in range(NSTEPS):
        s = i % NBUF
        in_copy(i, s).wait()
        if i >= NBUF:
            out_copy(i - NBUF, s).wait()
        obuf[s] = (
            jnp.dot(xbuf[s], wt_ref[...], preferred_element_type=jnp.float32)
            + b_ref[...]
        )
        out_copy(i, s).start()
        if i + NBUF < NSTEPS:
            in_copy(i + NBUF, s).start()
    for i in range(NSTEPS - NBUF, NSTEPS):
        out_copy(i, i % NBUF).wait()


@jax.jit
def _matmul(input, wt, bias2d):
    return pl.pallas_call(
        _mm_body,
        in_specs=[
            pl.BlockSpec(memory_space=pl.ANY),
            pl.BlockSpec(memory_space=pl.ANY),
            pl.BlockSpec(memory_space=pltpu.VMEM),
            pl.BlockSpec(memory_space=pltpu.VMEM),
        ],
        out_specs=pl.BlockSpec(memory_space=pl.ANY),
        out_shape=jax.ShapeDtypeStruct((N, M), jnp.float32),
        scratch_shapes=[
            pltpu.VMEM((NBUF, BLOCK_N, K), jnp.float32),
            pltpu.VMEM((NBUF, BLOCK_N, M), jnp.float32),
            pltpu.SemaphoreType.DMA((NBUF,)),
            pltpu.SemaphoreType.DMA((NBUF,)),
        ],
    )(input, input, wt, bias2d)


def kernel(input, weight, bias):
    return _matmul(input, weight.T, bias.reshape(1, M))


# speedup vs baseline: 1.0009x; 1.0009x over previous
import jax
import jax.numpy as jnp
from jax.experimental import pallas as pl
from jax.experimental.pallas import tpu as pltpu

N = 65536
K = 256
M = 64
BLOCK_N = 1024
NBUF = 8
NSTEPS = N // BLOCK_N


def _mm_body(xa_hbm, xb_hbm, wt_ref, b_ref, o_hbm, *rest):
    xrefs = [xa_hbm, xb_hbm]
    xbufs = rest[:NBUF]
    obufs = rest[NBUF:2 * NBUF]
    insems, outsems = rest[2 * NBUF], rest[2 * NBUF + 1]

    def in_copy(i, s):
        return pltpu.make_async_copy(
            xrefs[i % 2].at[pl.ds(i * BLOCK_N, BLOCK_N), :],
            xbufs[s],
            insems.at[s],
        )

    def out_copy(i, s):
        return pltpu.make_async_copy(
            obufs[s], o_hbm.at[pl.ds(i * BLOCK_N, BLOCK_N), :], outsems.at[s]
        )

    for i in range(NBUF):
        in_copy(i, i).start()
    for i in range(NSTEPS):
        s = i % NBUF
        in_copy(i, s).wait()
        if i >= NBUF:
            out_copy(i - NBUF, s).wait()
        obufs[s][...] = (
            jnp.dot(
                xbufs[s][...], wt_ref[...], preferred_element_type=jnp.float32
            )
            + b_ref[...]
        )
        out_copy(i, s).start()
        if i + NBUF < NSTEPS:
            in_copy(i + NBUF, s).start()
    for i in range(NSTEPS - NBUF, NSTEPS):
        out_copy(i, i % NBUF).wait()


@jax.jit
def _matmul(input, wt, bias2d):
    return pl.pallas_call(
        _mm_body,
        in_specs=[
            pl.BlockSpec(memory_space=pl.ANY),
            pl.BlockSpec(memory_space=pl.ANY),
            pl.BlockSpec(memory_space=pltpu.VMEM),
            pl.BlockSpec(memory_space=pltpu.VMEM),
        ],
        out_specs=pl.BlockSpec(memory_space=pl.ANY),
        out_shape=jax.ShapeDtypeStruct((N, M), jnp.float32),
        scratch_shapes=(
            [pltpu.VMEM((BLOCK_N, K), jnp.float32) for _ in range(NBUF)]
            + [pltpu.VMEM((BLOCK_N, M), jnp.float32) for _ in range(NBUF)]
            + [
                pltpu.SemaphoreType.DMA((NBUF,)),
                pltpu.SemaphoreType.DMA((NBUF,)),
            ]
        ),
    )(input, input, wt, bias2d)


def kernel(input, weight, bias):
    return _matmul(input, weight.T, bias.reshape(1, M))
